# baseline (speedup 1.0000x reference)
import functools

import jax
import jax.numpy as jnp
from jax import lax
from jax.experimental import pallas as pl
from jax.experimental.pallas import tpu as pltpu

_TRUNK = 128
_D = 64


def _tree_sum(xs):
    xs = list(xs)
    while len(xs) > 1:
        nxt = [xs[i] + xs[i + 1] for i in range(0, len(xs) - 1, 2)]
        if len(xs) % 2:
            nxt.append(xs[-1])
        xs = nxt
    return xs[0]


def _pass_a_kernel(k_ref, v_ref, m_ref, sp_ref, *, tb):
    b = pl.program_id(1)
    kk = k_ref[...].reshape(tb, _TRUNK, _D)
    vv = v_ref[...].reshape(tb, _TRUNK, _D)
    ms = []
    for t in range(tb):
        m_t = lax.dot_general(
            kk[t], vv[t], (((0,), (0,)), ((), ())),
            preferred_element_type=jnp.float32)
        m_ref[t] = m_t
        ms.append(m_t)
    s_blk = _tree_sum(ms)

    @pl.when(b == 0)
    def _():
        sp_ref[...] = jnp.zeros_like(sp_ref)

    sp_ref[...] += s_blk[None]


def _pass_b_kernel(q_ref, m_ref, sp_ref, o_ref, *, tb):
    ssum = jnp.sum(sp_ref[...], axis=0)
    qq = q_ref[...]
    mm = m_ref[...]
    for t in range(tb):
        b_t = lax.dot_general(
            mm[t], ssum, (((1,), (1,)), ((), ())),
            preferred_element_type=jnp.float32)
        o_ref[t * _TRUNK:(t + 1) * _TRUNK, :] = lax.dot_general(
            qq[t * _TRUNK:(t + 1) * _TRUNK, :], b_t,
            (((1,), (0,)), ((), ())),
            preferred_element_type=jnp.float32)


@jax.jit
def kernel(Q, K, V):
    N, D = Q.shape
    assert D == _D and N % _TRUNK == 0
    T = N // _TRUNK
    TB = 64
    P = 2
    assert T % TB == 0
    G = T // TB
    assert G % P == 0
    B1 = G // P
    R = TB * _TRUNK

    m_arr, s_parts = pl.pallas_call(
        functools.partial(_pass_a_kernel, tb=TB),
        grid=(P, B1),
        in_specs=[
            pl.BlockSpec((R, _D), lambda p, b: (p * B1 + b, 0)),
            pl.BlockSpec((R, _D), lambda p, b: (p * B1 + b, 0)),
        ],
        out_specs=[
            pl.BlockSpec((TB, _D, _D), lambda p, b: (p * B1 + b, 0, 0)),
            pl.BlockSpec((1, _D, _D), lambda p, b: (p, 0, 0)),
        ],
        out_shape=[
            jax.ShapeDtypeStruct((T, _D, _D), jnp.float32),
            jax.ShapeDtypeStruct((P, _D, _D), jnp.float32),
        ],
        compiler_params=pltpu.CompilerParams(
            dimension_semantics=("parallel", "arbitrary"),
        ),
        name="la_pass_a",
    )(K, V)

    out = pl.pallas_call(
        functools.partial(_pass_b_kernel, tb=TB),
        grid=(P, B1),
        in_specs=[
            pl.BlockSpec((R, _D), lambda p, b: (p * B1 + b, 0)),
            pl.BlockSpec((TB, _D, _D), lambda p, b: (p * B1 + b, 0, 0)),
            pl.BlockSpec((P, _D, _D), lambda p, b: (0, 0, 0)),
        ],
        out_specs=pl.BlockSpec((R, _D), lambda p, b: (p * B1 + b, 0)),
        out_shape=jax.ShapeDtypeStruct((N, _D), jnp.float32),
        compiler_params=pltpu.CompilerParams(
            dimension_semantics=("parallel", "arbitrary"),
        ),
        name="la_pass_b",
    )(Q, m_arr, s_parts)

    return out


# speedup vs baseline: 1.4044x; 1.4044x over previous
import functools

import jax
import jax.numpy as jnp
from jax import lax
from jax.experimental import pallas as pl
from jax.experimental.pallas import tpu as pltpu

_TRUNK = 128
_D = 64
_NACC = 4


def _pass_a_kernel(k_ref, v_ref, m_ref, sp_ref, *, tb):
    b = pl.program_id(1)
    accs = [None] * _NACC
    for t in range(tb):
        sl = slice(t * _TRUNK, (t + 1) * _TRUNK)
        kt = k_ref[sl, :].astype(jnp.bfloat16)
        vt = v_ref[sl, :].astype(jnp.bfloat16)
        m_t = lax.dot_general(
            kt, vt, (((0,), (0,)), ((), ())),
            preferred_element_type=jnp.float32)
        m_ref[t] = m_t.astype(jnp.bfloat16)
        a = t % _NACC
        accs[a] = m_t if accs[a] is None else accs[a] + m_t
    s_blk = accs[0]
    for a in range(1, _NACC):
        s_blk = s_blk + accs[a]

    @pl.when(b == 0)
    def _():
        sp_ref[...] = jnp.zeros_like(sp_ref)

    sp_ref[...] += s_blk[None]


def _pass_b_kernel(q_ref, m_ref, sp_ref, o_ref, p_scr, *, tb):
    ssum = jnp.sum(sp_ref[...], axis=0)
    ssb = ssum.astype(jnp.bfloat16)
    mm = m_ref[...].reshape(tb * _D, _D)
    p = lax.dot_general(
        mm, ssb, (((1,), (1,)), ((), ())),
        preferred_element_type=jnp.float32)
    p_scr[...] = p.astype(jnp.bfloat16)
    for t in range(tb):
        sl = slice(t * _TRUNK, (t + 1) * _TRUNK)
        qt = q_ref[sl, :].astype(jnp.bfloat16)
        pt = p_scr[t * _D:(t + 1) * _D, :]
        o_ref[sl, :] = lax.dot_general(
            qt, pt, (((1,), (0,)), ((), ())),
            preferred_element_type=jnp.float32)


@jax.jit
def kernel(Q, K, V):
    N, D = Q.shape
    assert D == _D and N % _TRUNK == 0
    T = N // _TRUNK
    TB = 64
    P = 2
    assert T % TB == 0
    G = T // TB
    assert G % P == 0
    B1 = G // P
    R = TB * _TRUNK

    m_arr, s_parts = pl.pallas_call(
        functools.partial(_pass_a_kernel, tb=TB),
        grid=(P, B1),
        in_specs=[
            pl.BlockSpec((R, _D), lambda p, b: (p * B1 + b, 0)),
            pl.BlockSpec((R, _D), lambda p, b: (p * B1 + b, 0)),
        ],
        out_specs=[
            pl.BlockSpec((TB, _D, _D), lambda p, b: (p * B1 + b, 0, 0)),
            pl.BlockSpec((1, _D, _D), lambda p, b: (p, 0, 0)),
        ],
        out_shape=[
            jax.ShapeDtypeStruct((T, _D, _D), jnp.bfloat16),
            jax.ShapeDtypeStruct((P, _D, _D), jnp.float32),
        ],
        compiler_params=pltpu.CompilerParams(
            dimension_semantics=("parallel", "arbitrary"),
        ),
        name="la_pass_a",
    )(K, V)

    out = pl.pallas_call(
        functools.partial(_pass_b_kernel, tb=TB),
        grid=(P, B1),
        in_specs=[
            pl.BlockSpec((R, _D), lambda p, b: (p * B1 + b, 0)),
            pl.BlockSpec((TB, _D, _D), lambda p, b: (p * B1 + b, 0, 0)),
            pl.BlockSpec((P, _D, _D), lambda p, b: (0, 0, 0)),
        ],
        out_specs=pl.BlockSpec((R, _D), lambda p, b: (p * B1 + b, 0)),
        out_shape=jax.ShapeDtypeStruct((N, _D), jnp.float32),
        scratch_shapes=[pltpu.VMEM((TB * _D, _D), jnp.bfloat16)],
        compiler_params=pltpu.CompilerParams(
            dimension_semantics=("parallel", "arbitrary"),
        ),
        name="la_pass_b",
    )(Q, m_arr, s_parts)

    return out
